# baseline (speedup 1.0000x reference)
import functools

import jax
import jax.numpy as jnp
from jax import lax
from jax.experimental import pallas as pl
from jax.experimental.pallas import tpu as pltpu
from jax.experimental.pallas import tpu_sc as plsc

VOCAB = 1000000
EMB = 64
MAX_LEN = 200
CLASSES = 100
BATCH = 4096

NC = 2
NS = 16
NW = NC * NS
B_PER_W = BATCH // NW
TOK_PER_W = B_PER_W * MAX_LEN
PAIR_ROWS = VOCAB // 2
S0 = 128
S1 = MAX_LEN - S0


def _pool_body(rows_hbm, hoff_hbm, table_hbm, out_hbm,
               rows_v, hoff_v, buf, stage, sem):
    wid = lax.axis_index("s") * NC + lax.axis_index("c")
    base = wid * B_PER_W

    pltpu.sync_copy(rows_hbm.at[pl.ds(wid * TOK_PER_W, TOK_PER_W)], rows_v)
    pltpu.sync_copy(hoff_hbm.at[pl.ds(wid * TOK_PER_W, TOK_PER_W)],
                    hoff_v.at[pl.ds(0, TOK_PER_W)])

    zero = jnp.zeros((16,), jnp.float32)

    def item_body(i, _):
        tok = i * MAX_LEN
        c0 = pltpu.async_copy(
            table_hbm.at[rows_v.at[pl.ds(tok, S0)]],
            buf.at[pl.ds(0, S0)], sem)
        c1 = pltpu.async_copy(
            table_hbm.at[rows_v.at[pl.ds(tok + S0, S1)]],
            buf.at[pl.ds(S0, S1)], sem)
        c0.wait()
        c1.wait()

        def red16(m, accs):
            hv = hoff_v[pl.ds(tok + 16 * m, 16)]
            base_l = 16 * m
            for j in range(16):
                h = hv[j]
                accs = tuple(
                    accs[k] + buf[base_l + j, pl.ds(h + 16 * k, 16)]
                    for k in range(4)
                )
            return accs
        accs = lax.fori_loop(0, MAX_LEN // 16, red16,
                             (zero, zero, zero, zero))
        hv = hoff_v[pl.ds(tok + MAX_LEN - MAX_LEN % 16, 16)]
        accs = list(accs)
        for j in range(MAX_LEN % 16):
            h = hv[j]
            for k in range(4):
                accs[k] = accs[k] + buf[MAX_LEN - MAX_LEN % 16 + j,
                                        pl.ds(h + 16 * k, 16)]
        for k in range(4):
            stage[i, pl.ds(16 * k, 16)] = accs[k]
        return 0

    lax.fori_loop(0, B_PER_W, item_body, 0)
    pltpu.sync_copy(stage, out_hbm.at[pl.ds(base, B_PER_W)])


_pool_call = functools.partial(
    pl.kernel,
    out_type=jax.ShapeDtypeStruct((BATCH, EMB), jnp.float32),
    mesh=plsc.VectorSubcoreMesh(core_axis_name="c", subcore_axis_name="s"),
    scratch_types=[
        pltpu.VMEM((TOK_PER_W,), jnp.int32),
        pltpu.VMEM((TOK_PER_W + 16,), jnp.int32),
        pltpu.VMEM((MAX_LEN, 2 * EMB), jnp.float32),
        pltpu.VMEM((B_PER_W, EMB), jnp.float32),
        pltpu.SemaphoreType.DMA,
    ],
)(_pool_body)


CPAD = 128
_DBLK = 512


def _dense_kernel(x_ref, w_ref, b_ref, o_ref):
    logits = jnp.dot(x_ref[...], w_ref[...],
                     preferred_element_type=jnp.float32) + b_ref[...]
    m = jnp.max(logits, axis=-1, keepdims=True)
    e = jnp.exp(logits - m)
    o_ref[...] = e / jnp.sum(e, axis=-1, keepdims=True)


_dense_call = pl.pallas_call(
    _dense_kernel,
    grid=(BATCH // _DBLK,),
    in_specs=[
        pl.BlockSpec((_DBLK, EMB), lambda i: (i, 0)),
        pl.BlockSpec((EMB, CPAD), lambda i: (0, 0)),
        pl.BlockSpec((1, CPAD), lambda i: (0, 0)),
    ],
    out_specs=pl.BlockSpec((_DBLK, CPAD), lambda i: (i, 0)),
    out_shape=jax.ShapeDtypeStruct((BATCH, CPAD), jnp.float32),
)


def kernel(inputs, table, W, b):
    idx = inputs.astype(jnp.int32).reshape(-1)
    rows = idx >> 1
    hoff = (idx & 1) << 6
    table2 = table.reshape(PAIR_ROWS, 2 * EMB)
    pool_sum = _pool_call(rows, hoff, table2)
    w_pad = jnp.pad(W * (1.0 / MAX_LEN), ((0, 0), (0, CPAD - CLASSES)))
    b_pad = jnp.concatenate(
        [b, jnp.full((CPAD - CLASSES,), -1e30, b.dtype)]).reshape(1, CPAD)
    out = _dense_call(pool_sum, w_pad, b_pad)
    return out[:, :CLASSES]
